# baseline (speedup 1.0000x reference)
import functools

import jax
import jax.numpy as jnp
from jax import lax
from jax.experimental import pallas as pl
from jax.experimental.pallas import tpu as pltpu
from jax.experimental.pallas import tpu_sc as plsc

V = 100000
D = 64
B = 16384
L = 50
K = 5
ALPHA = 0.5

NC = 2
NS = 16
NW = NC * NS
BPW = B // NW
CB = 16
NCHUNK = BPW // CB

LP = 64
KP = 16

def _bcast_lane(vec, lane):
    idx = jnp.full((16, 1), lane, dtype=jnp.int32)
    dnums = lax.GatherDimensionNumbers(
        offset_dims=(), collapsed_slice_dims=(0,), start_index_map=(0,))
    return lax.gather(vec, idx, dnums, slice_sizes=(1,),
                      mode=lax.GatherScatterMode.PROMISE_IN_BOUNDS)


def _sc_body(tgt_ref, ctx_ref, negi_ref, wf_ref, wr_ref, wiso_ref, scal_ref,
             pos_out, neg_out, *scr):
    bufs = (scr[0:9], scr[9:18])
    pos_sv, neg_sv = scr[18], scr[19]
    sems = (scr[20], scr[21])

    wid = lax.axis_index("s") * NC + lax.axis_index("c")
    row0 = wid * BPW

    def copies(p, base):
        tgt, ctxi, negi, fwd, rev, iso, scal, ctxr, negr = bufs[p]
        return (
            (wf_ref.at[tgt], fwd),
            (wr_ref.at[tgt], rev),
            (wiso_ref.at[tgt], iso),
            (scal_ref.at[tgt], scal),
            (wf_ref.at[ctxi], ctxr),
            (wf_ref.at[negi], negr),
        )

    def stage_and_fire(ci, p):
        nbase = row0 + ci * CB
        tgt, ctxi, negi = bufs[p][0], bufs[p][1], bufs[p][2]
        pltpu.sync_copy(tgt_ref.at[pl.ds(nbase, CB)], tgt)
        pltpu.sync_copy(ctx_ref.at[pl.ds(nbase * L, CB * L)], ctxi)
        pltpu.sync_copy(negi_ref.at[pl.ds(nbase * K, CB * K)], negi)
        for src, dst in copies(p, nbase):
            pltpu.async_copy(src, dst, sems[p])

    def drain(p):
        for src, dst in copies(p, 0):
            pltpu.make_async_copy(src, dst, sems[p]).wait()

    def compute(ci, p):
        base = row0 + ci * CB
        _, _, _, fwd, rev, iso, scal, ctxr, negr = bufs[p]
        sc_all = scal[pl.ds(0, 16)]
        iota16 = jnp.arange(16, dtype=jnp.int32)

        def b_body(b, _):
            scb = _bcast_lane(sc_all, b)
            og = []
            for g in range(4):
                f = fwd[b, pl.ds(g * 16, 16)]
                r = rev[b, pl.ds(g * 16, 16)]
                s = iso[b, pl.ds(g * 16, 16)]
                og.append(f * ALPHA + r * (1.0 - ALPHA) + s * scb)

            rowvs = []
            for g in range(4):
                rv = iota16 + (b * L + g * 16)
                rowvs.append((jnp.minimum(rv, CB * L - 1), ctxr))
            rowvs.append((jnp.minimum(iota16 + b * K, CB * K - 1), negr))

            accs = [jnp.zeros((16,), jnp.float32) for _ in range(5)]
            for g16 in range(4):
                for c in range(16):
                    ob = _bcast_lane(og[g16], c)
                    cold = jnp.full((16,), g16 * 16 + c, dtype=jnp.int32)
                    for gi, (rv, rref) in enumerate(rowvs):
                        val = plsc.load_gather(rref, [rv, cold])
                        accs[gi] = accs[gi] + ob * val
            for g in range(4):
                pos_sv[pl.ds(b * LP + g * 16, 16)] = accs[g]
            neg_sv[pl.ds(b * KP, 16)] = accs[4]
            return _

        lax.fori_loop(0, CB, b_body, None)
        pltpu.sync_copy(pos_sv, pos_out.at[pl.ds(base * LP, CB * LP)])
        pltpu.sync_copy(neg_sv, neg_out.at[pl.ds(base * KP, CB * KP)])

    stage_and_fire(0, 0)

    def outer_body(co, carry):
        for p in range(2):
            ci = co * 2 + p

            @pl.when(ci + 1 < NCHUNK)
            def _():
                stage_and_fire(ci + 1, 1 - p)

            drain(p)
            compute(ci, p)
        return carry

    lax.fori_loop(0, NCHUNK // 2, outer_body, None)


def _buf_set():
    return [
        pltpu.VMEM((CB,), jnp.int32),
        pltpu.VMEM((CB * L,), jnp.int32),
        pltpu.VMEM((CB * K,), jnp.int32),
        pltpu.VMEM((CB, D), jnp.float32),
        pltpu.VMEM((CB, D), jnp.float32),
        pltpu.VMEM((CB, D), jnp.float32),
        pltpu.VMEM((CB,), jnp.float32),
        pltpu.VMEM((CB * L, D), jnp.float32),
        pltpu.VMEM((CB * K, D), jnp.float32),
    ]


_sc_scores = functools.partial(
    pl.kernel,
    out_type=(
        jax.ShapeDtypeStruct((B * LP,), jnp.float32),
        jax.ShapeDtypeStruct((B * KP,), jnp.float32),
    ),
    mesh=plsc.VectorSubcoreMesh(
        core_axis_name="c", subcore_axis_name="s",
        num_cores=NC, num_subcores=NS),
    compiler_params=pltpu.CompilerParams(
        needs_layout_passes=False, use_tc_tiling_on_sc=False),
    scratch_types=_buf_set() + _buf_set() + [
        pltpu.VMEM((CB * LP,), jnp.float32),
        pltpu.VMEM((CB * KP,), jnp.float32),
        pltpu.SemaphoreType.DMA,
        pltpu.SemaphoreType.DMA,
    ],
)(_sc_body)


def _loss_body(pos_ref, neg_ref, out_ref):
    pos = pos_ref[...]
    lane_p = lax.broadcasted_iota(jnp.int32, (B * LP,), 0) % LP
    pt = -jnp.log(jax.nn.sigmoid(pos) + 1e-6)
    psum = jnp.sum(jnp.where(lane_p < L, pt, 0.0))

    neg = neg_ref[...]
    lane_n = lax.broadcasted_iota(jnp.int32, (B * KP,), 0) % KP
    nt = -jnp.log(jax.nn.sigmoid(-neg) + 1e-6)
    nsum = jnp.sum(jnp.where(lane_n < K, nt, 0.0))

    out_ref[0, 0] = psum / (B * float(L)) + nsum / float(K)


_loss_tc = pl.pallas_call(
    _loss_body,
    out_shape=jax.ShapeDtypeStruct((1, 1), jnp.float32),
    out_specs=pl.BlockSpec(memory_space=pltpu.SMEM),
)


def kernel(target_tokens, context_tokens, neg_idx, W_fwd, W_rev, W_iso,
           token_frequencies):
    tgt = target_tokens.astype(jnp.int32)
    ctx = context_tokens.astype(jnp.int32).reshape(B * L)
    neg = neg_idx.astype(jnp.int32).reshape(B * K)
    scal = 1.0 / (1.0 + jnp.log(token_frequencies + 1e-6))
    pos_s, neg_s = _sc_scores(tgt, ctx, neg, W_fwd, W_rev, W_iso, scal)
    return _loss_tc(pos_s, neg_s)[0, 0]


# speedup vs baseline: 3.8607x; 3.8607x over previous
import functools

import jax
import jax.numpy as jnp
from jax import lax
from jax.experimental import pallas as pl
from jax.experimental.pallas import tpu as pltpu
from jax.experimental.pallas import tpu_sc as plsc

V = 100000
D = 64
B = 16384
L = 50
K = 5
ALPHA = 0.5

NC = 2
NS = 16
NW = NC * NS
BPW = B // NW
CB = 16
NCHUNK = BPW // CB

LP = 64
KP = 16

def _perm(vec, idx):
    dnums = lax.GatherDimensionNumbers(
        offset_dims=(), collapsed_slice_dims=(0,), start_index_map=(0,))
    return lax.gather(vec, idx[:, None], dnums, slice_sizes=(1,),
                      mode=lax.GatherScatterMode.PROMISE_IN_BOUNDS)


def _bcast_lane(vec, lane):
    return _perm(vec, jnp.full((16,), lane, dtype=jnp.int32))


def _sc_body(tgt_ref, ctx_ref, negi_ref, wf_ref, wr_ref, wiso_ref, scal_ref,
             pos_out, neg_out, *scr):
    bufs = (scr[0:9], scr[9:18])
    pos_sv, neg_sv = scr[18], scr[19]
    sems = (scr[20], scr[21])

    wid = lax.axis_index("s") * NC + lax.axis_index("c")
    row0 = wid * BPW

    def copies(p, base):
        tgt, ctxi, negi, fwd, rev, iso, scal, ctxr, negr = bufs[p]
        return (
            (wf_ref.at[tgt], fwd),
            (wr_ref.at[tgt], rev),
            (wiso_ref.at[tgt], iso),
            (scal_ref.at[tgt], scal),
            (wf_ref.at[ctxi], ctxr),
            (wf_ref.at[negi], negr),
        )

    def stage_and_fire(ci, p):
        nbase = row0 + ci * CB
        tgt, ctxi, negi = bufs[p][0], bufs[p][1], bufs[p][2]
        pltpu.sync_copy(tgt_ref.at[pl.ds(nbase, CB)], tgt)
        pltpu.sync_copy(ctx_ref.at[pl.ds(nbase * L, CB * L)], ctxi)
        pltpu.sync_copy(negi_ref.at[pl.ds(nbase * K, CB * K)], negi)
        for src, dst in copies(p, nbase):
            pltpu.async_copy(src, dst, sems[p])

    def drain(p):
        for src, dst in copies(p, 0):
            pltpu.make_async_copy(src, dst, sems[p]).wait()

    def compute(ci, p):
        base = row0 + ci * CB
        _, _, _, fwd, rev, iso, scal, ctxr, negr = bufs[p]
        sc_all = scal[pl.ds(0, 16)]
        iota16 = jnp.arange(16, dtype=jnp.int32)
        bfly = [iota16 ^ 1, iota16 ^ 2, iota16 ^ 4, iota16 ^ 8]
        masks = [iota16 == j for j in range(16)]

        def b_body(b, _):
            scb = _bcast_lane(sc_all, b)
            og = []
            for g in range(4):
                f = fwd[b, pl.ds(g * 16, 16)]
                r = rev[b, pl.ds(g * 16, 16)]
                s = iso[b, pl.ds(g * 16, 16)]
                og.append(f * ALPHA + r * (1.0 - ALPHA) + s * scb)

            def dot_all(rows_ref, r):
                t = og[0] * rows_ref[r, pl.ds(0, 16)]
                for g in range(1, 4):
                    t = t + og[g] * rows_ref[r, pl.ds(g * 16, 16)]
                for s in bfly:
                    t = t + _perm(t, s)
                return t

            for g in range(4):
                nl = min(16, L - g * 16)
                sv = jnp.zeros((16,), jnp.float32)
                for j in range(nl):
                    t = dot_all(ctxr, b * L + g * 16 + j)
                    sv = jnp.where(masks[j], t, sv)
                pos_sv[pl.ds(b * LP + g * 16, 16)] = sv

            sv = jnp.zeros((16,), jnp.float32)
            for k in range(K):
                t = dot_all(negr, b * K + k)
                sv = jnp.where(masks[k], t, sv)
            neg_sv[pl.ds(b * KP, 16)] = sv
            return _

        lax.fori_loop(0, CB, b_body, None)
        pltpu.sync_copy(pos_sv, pos_out.at[pl.ds(base * LP, CB * LP)])
        pltpu.sync_copy(neg_sv, neg_out.at[pl.ds(base * KP, CB * KP)])

    stage_and_fire(0, 0)

    def outer_body(co, carry):
        for p in range(2):
            ci = co * 2 + p

            @pl.when(ci + 1 < NCHUNK)
            def _():
                stage_and_fire(ci + 1, 1 - p)

            drain(p)
            compute(ci, p)
        return carry

    lax.fori_loop(0, NCHUNK // 2, outer_body, None)


def _buf_set():
    return [
        pltpu.VMEM((CB,), jnp.int32),
        pltpu.VMEM((CB * L,), jnp.int32),
        pltpu.VMEM((CB * K,), jnp.int32),
        pltpu.VMEM((CB, D), jnp.float32),
        pltpu.VMEM((CB, D), jnp.float32),
        pltpu.VMEM((CB, D), jnp.float32),
        pltpu.VMEM((CB,), jnp.float32),
        pltpu.VMEM((CB * L, D), jnp.float32),
        pltpu.VMEM((CB * K, D), jnp.float32),
    ]


_sc_scores = functools.partial(
    pl.kernel,
    out_type=(
        jax.ShapeDtypeStruct((B * LP,), jnp.float32),
        jax.ShapeDtypeStruct((B * KP,), jnp.float32),
    ),
    mesh=plsc.VectorSubcoreMesh(
        core_axis_name="c", subcore_axis_name="s",
        num_cores=NC, num_subcores=NS),
    compiler_params=pltpu.CompilerParams(
        needs_layout_passes=False, use_tc_tiling_on_sc=False),
    scratch_types=_buf_set() + _buf_set() + [
        pltpu.VMEM((CB * LP,), jnp.float32),
        pltpu.VMEM((CB * KP,), jnp.float32),
        pltpu.SemaphoreType.DMA,
        pltpu.SemaphoreType.DMA,
    ],
)(_sc_body)


def _loss_body(pos_ref, neg_ref, out_ref):
    pos = pos_ref[...]
    lane_p = lax.broadcasted_iota(jnp.int32, (B * LP,), 0) % LP
    pt = -jnp.log(jax.nn.sigmoid(pos) + 1e-6)
    psum = jnp.sum(jnp.where(lane_p < L, pt, 0.0))

    neg = neg_ref[...]
    lane_n = lax.broadcasted_iota(jnp.int32, (B * KP,), 0) % KP
    nt = -jnp.log(jax.nn.sigmoid(-neg) + 1e-6)
    nsum = jnp.sum(jnp.where(lane_n < K, nt, 0.0))

    out_ref[0, 0] = psum / (B * float(L)) + nsum / float(K)


_loss_tc = pl.pallas_call(
    _loss_body,
    out_shape=jax.ShapeDtypeStruct((1, 1), jnp.float32),
    out_specs=pl.BlockSpec(memory_space=pltpu.SMEM),
)


def kernel(target_tokens, context_tokens, neg_idx, W_fwd, W_rev, W_iso,
           token_frequencies):
    tgt = target_tokens.astype(jnp.int32)
    ctx = context_tokens.astype(jnp.int32).reshape(B * L)
    neg = neg_idx.astype(jnp.int32).reshape(B * K)
    scal = 1.0 / (1.0 + jnp.log(token_frequencies + 1e-6))
    pos_s, neg_s = _sc_scores(tgt, ctx, neg, W_fwd, W_rev, W_iso, scal)
    return _loss_tc(pos_s, neg_s)[0, 0]
